# baseline (speedup 1.0000x reference)
import functools
import math

import jax
import jax.numpy as jnp
from jax import lax
from jax.experimental import pallas as pl
from jax.experimental.pallas import tpu as pltpu
from jax.experimental.pallas import tpu_sc as plsc

N = 10000
E = 320000
G = 64
DIN = 128
H = 64
NC = 2
NS = 16
NW = NC * NS
CH = 80
NSTEP = 128
EPAD = NW * NSTEP * CH
PADROW = N + 8
ROWS_T = 632
NPAD = ROWS_T * NS
BN = 2000
NB = N // BN
K1 = 1.0 / math.sqrt(1.0 + 1e-5)

_MESH = plsc.VectorSubcoreMesh(core_axis_name="c", subcore_axis_name="s")
_SC_PARAMS = pltpu.CompilerParams(use_tc_tiling_on_sc=False)


@functools.partial(
    pl.kernel,
    out_type=jax.ShapeDtypeStruct((NC, NPAD, 16), jnp.float32),
    mesh=_MESH,
    scratch_types=[
        pltpu.VMEM((NSTEP, CH), jnp.int32),
        pltpu.VMEM((CH, 16), jnp.float32),
        pltpu.VMEM((ROWS_T, 16), jnp.float32),
        pltpu.VMEM_SHARED((NPAD, 16), jnp.float32),
    ],
    compiler_params=_SC_PARAMS,
)
def _sc_degree(dst_hbm, out_hbm, dst_v, ones_v, zero_v, acc):
    c = lax.axis_index("c")
    s = lax.axis_index("s")
    wid = c * NS + s

    def _ones(r, carry):
        ones_v[r, pl.ds(0, 16)] = jnp.ones((16,), jnp.float32)
        return carry

    def _zero(r, carry):
        zero_v[r, pl.ds(0, 16)] = jnp.zeros((16,), jnp.float32)
        return carry

    lax.fori_loop(0, CH, _ones, 0)
    lax.fori_loop(0, ROWS_T, _zero, 0)
    pltpu.sync_copy(zero_v, acc.at[pl.ds(s * ROWS_T, ROWS_T)])
    pltpu.sync_copy(dst_hbm.at[wid], dst_v)
    plsc.subcore_barrier()

    def _step(i, carry):
        pltpu.sync_copy(ones_v, acc.at[dst_v.at[i]], add=True)
        return carry

    lax.fori_loop(0, NSTEP, _step, 0)
    plsc.subcore_barrier()
    pltpu.sync_copy(acc.at[pl.ds(s * ROWS_T, ROWS_T)],
                    out_hbm.at[c, pl.ds(s * ROWS_T, ROWS_T)])


@functools.partial(
    pl.kernel,
    out_type=jax.ShapeDtypeStruct((NC, NPAD, H), jnp.float32),
    mesh=_MESH,
    scratch_types=[
        pltpu.VMEM((NSTEP, CH), jnp.int32),
        pltpu.VMEM((NSTEP, CH), jnp.int32),
        pltpu.VMEM((CH, H), jnp.float32),
        pltpu.VMEM((CH, H), jnp.float32),
        pltpu.VMEM((CH, H), jnp.float32),
        pltpu.VMEM((CH, H), jnp.float32),
        pltpu.VMEM((ROWS_T, H), jnp.float32),
        pltpu.VMEM_SHARED((NPAD, H), jnp.float32),
        pltpu.SemaphoreType.DMA,
        pltpu.SemaphoreType.DMA,
        pltpu.SemaphoreType.DMA,
        pltpu.SemaphoreType.DMA,
        pltpu.SemaphoreType.DMA,
        pltpu.SemaphoreType.DMA,
        pltpu.SemaphoreType.DMA,
        pltpu.SemaphoreType.DMA,
    ],
    compiler_params=_SC_PARAMS,
)
def _sc_gather_scatter(y_hbm, src_hbm, dst_hbm, out_hbm,
                       src_v, dst_v, r0, r1, r2, r3, zero_v, acc,
                       g0, g1, g2, g3, s0, s1, s2, s3):
    rows = (r0, r1, r2, r3)
    gsem = (g0, g1, g2, g3)
    ssem = (s0, s1, s2, s3)
    c = lax.axis_index("c")
    s = lax.axis_index("s")
    wid = c * NS + s

    def _zero(r, carry):
        for k in range(H // 16):
            zero_v[r, pl.ds(16 * k, 16)] = jnp.zeros((16,), jnp.float32)
        return carry

    lax.fori_loop(0, ROWS_T, _zero, 0)
    pltpu.sync_copy(zero_v, acc.at[pl.ds(s * ROWS_T, ROWS_T)])
    pltpu.sync_copy(src_hbm.at[wid], src_v)
    pltpu.sync_copy(dst_hbm.at[wid], dst_v)
    plsc.subcore_barrier()

    def _quad(k, carry):
        i = 4 * k
        gd = [pltpu.async_copy(y_hbm.at[src_v.at[i + b]], rows[b], gsem[b])
              for b in range(4)]
        sd = []
        for b in range(4):
            gd[b].wait()
            sd.append(pltpu.async_copy(rows[b], acc.at[dst_v.at[i + b]],
                                       ssem[b], add=True))
        for b in range(4):
            sd[b].wait()
        return carry

    lax.fori_loop(0, NSTEP // 4, _quad, 0)
    plsc.subcore_barrier()
    pltpu.sync_copy(acc.at[pl.ds(s * ROWS_T, ROWS_T)],
                    out_hbm.at[c, pl.ds(s * ROWS_T, ROWS_T)])


def _tc_pre_body(deg_ref, x_ref, w_ref, y_ref, dinv_ref):
    deg = deg_ref[0] + deg_ref[1] + 1.0
    dinv = lax.rsqrt(deg[:, :1])
    xw = jnp.dot(x_ref[...], w_ref[...], preferred_element_type=jnp.float32)
    y_ref[...] = xw * dinv
    dinv_ref[...] = jnp.broadcast_to(dinv, (BN, H))


def _tc_pre(deg, x, w1):
    return pl.pallas_call(
        _tc_pre_body,
        grid=(NB,),
        in_specs=[
            pl.BlockSpec((NC, BN, 16), lambda i: (0, i, 0)),
            pl.BlockSpec((BN, DIN), lambda i: (i, 0)),
            pl.BlockSpec((DIN, H), lambda i: (0, 0)),
        ],
        out_specs=[
            pl.BlockSpec((BN, H), lambda i: (i, 0)),
            pl.BlockSpec((BN, H), lambda i: (i, 0)),
        ],
        out_shape=[
            jax.ShapeDtypeStruct((N, H), jnp.float32),
            jax.ShapeDtypeStruct((N, H), jnp.float32),
        ],
    )(deg, x, w1)


def _make_tc_layer(has_res):
    def body(parts_ref, y_ref, dinv_ref, *rest):
        if has_res:
            res_ref, b_ref, g_ref, be_ref, w_ref, x_out, y_out = rest
        else:
            b_ref, g_ref, be_ref, w_ref, x_out, y_out = rest
        dinv = dinv_ref[...]
        agg = (parts_ref[0] + parts_ref[1] + y_ref[...]) * dinv
        h = (agg + b_ref[...]) * K1 * g_ref[...] + be_ref[...]
        xl = jnp.maximum(h, 0.0)
        if has_res:
            xl = xl + res_ref[...]
        x_out[...] = xl
        y_out[...] = jnp.dot(xl, w_ref[...],
                             preferred_element_type=jnp.float32) * dinv

    node = pl.BlockSpec((BN, H), lambda i: (i, 0))
    row = pl.BlockSpec((1, H), lambda i: (0, 0))
    in_specs = [pl.BlockSpec((NC, BN, H), lambda i: (0, i, 0)), node, node]
    if has_res:
        in_specs.append(node)
    in_specs += [row, row, row, pl.BlockSpec((H, H), lambda i: (0, 0))]

    def run(*args):
        return pl.pallas_call(
            body,
            grid=(NB,),
            in_specs=in_specs,
            out_specs=[node, node],
            out_shape=[
                jax.ShapeDtypeStruct((N, H), jnp.float32),
                jax.ShapeDtypeStruct((N, H), jnp.float32),
            ],
        )(*args)

    return run


_tc_layer_nores = _make_tc_layer(False)
_tc_layer_res = _make_tc_layer(True)


def _tc_final_body(parts_ref, y_ref, dinv_ref, b_ref, batch_ref,
                   wl1_ref, bl1_ref, wl2_ref, bl2_ref, out_ref, acc):
    i = pl.program_id(0)
    x4 = (parts_ref[0] + parts_ref[1] + y_ref[...]) * dinv_ref[...] + b_ref[...]
    bvec = batch_ref[0, 0, :]
    oht = (lax.broadcasted_iota(jnp.int32, (G, BN), 0)
           == bvec[None, :]).astype(jnp.float32)
    aug = jnp.concatenate([x4, jnp.ones_like(x4)], axis=1)
    part = jnp.dot(oht, aug, preferred_element_type=jnp.float32)
    acc[...] = jnp.where(i == 0, part, acc[...] + part)

    @pl.when(i == NB - 1)
    def _():
        sums = acc[:, :H]
        cnt = acc[:, H:H + 1]
        pooled = sums / jnp.maximum(cnt, 1.0)
        hmid = jnp.maximum(
            jnp.dot(pooled, wl1_ref[...], preferred_element_type=jnp.float32)
            + bl1_ref[...], 0.0)
        logits = jnp.dot(hmid, wl2_ref[...],
                         preferred_element_type=jnp.float32) + bl2_ref[...]
        m = jnp.max(logits, axis=1, keepdims=True)
        lse = m + jnp.log(jnp.sum(jnp.exp(logits - m), axis=1, keepdims=True))
        out_ref[...] = logits - lse


def _tc_final(parts, y4, dinv, b4, batch3, wl1, bl1, wl2, bl2):
    node = pl.BlockSpec((BN, H), lambda i: (i, 0))
    return pl.pallas_call(
        _tc_final_body,
        grid=(NB,),
        in_specs=[
            pl.BlockSpec((NC, BN, H), lambda i: (0, i, 0)),
            node, node,
            pl.BlockSpec((1, H), lambda i: (0, 0)),
            pl.BlockSpec((1, 1, BN), lambda i: (i, 0, 0)),
            pl.BlockSpec((H, H // 2), lambda i: (0, 0)),
            pl.BlockSpec((1, H // 2), lambda i: (0, 0)),
            pl.BlockSpec((H // 2, 2), lambda i: (0, 0)),
            pl.BlockSpec((1, 2), lambda i: (0, 0)),
        ],
        out_specs=pl.BlockSpec((G, 2), lambda i: (0, 0)),
        out_shape=jax.ShapeDtypeStruct((G, 2), jnp.float32),
        scratch_shapes=[pltpu.VMEM((G, 2 * H), jnp.float32)],
    )(parts, y4, dinv, b4, batch3, wl1, bl1, wl2, bl2)


def kernel(x, edge_index, batch, W1, b1, W2, b2, W3, b3, W4, b4,
           g1, be1, g2, be2, g3, be3, Wl1, bl1, Wl2, bl2):
    npad_e = EPAD - E
    src = jnp.concatenate(
        [edge_index[0].astype(jnp.int32), jnp.zeros((npad_e,), jnp.int32)]
    ).reshape(NW, NSTEP, CH)
    dst_pad = N + jnp.arange(npad_e, dtype=jnp.int32) % (NPAD - N)
    dst = jnp.concatenate(
        [edge_index[1].astype(jnp.int32), dst_pad]
    ).reshape(NW, NSTEP, CH)
    batch3 = batch.astype(jnp.int32).reshape(NB, 1, BN)
    b1r, g1r, be1r = b1.reshape(1, H), g1.reshape(1, H), be1.reshape(1, H)
    b2r, g2r, be2r = b2.reshape(1, H), g2.reshape(1, H), be2.reshape(1, H)
    b3r, g3r, be3r = b3.reshape(1, H), g3.reshape(1, H), be3.reshape(1, H)
    b4r = b4.reshape(1, H)

    deg = _sc_degree(dst)[:, :N]
    y1, dinv = _tc_pre(deg, x, W1)
    p1 = _sc_gather_scatter(y1, src, dst)[:, :N]
    x1, y2 = _tc_layer_nores(p1, y1, dinv, b1r, g1r, be1r, W2)
    p2 = _sc_gather_scatter(y2, src, dst)[:, :N]
    x2, y3 = _tc_layer_res(p2, y2, dinv, x1, b2r, g2r, be2r, W3)
    p3 = _sc_gather_scatter(y3, src, dst)[:, :N]
    x3, y4 = _tc_layer_res(p3, y3, dinv, x2, b3r, g3r, be3r, W4)
    p4 = _sc_gather_scatter(y4, src, dst)[:, :N]
    return _tc_final(p4, y4, dinv, b4r, batch3,
                     Wl1, bl1.reshape(1, H // 2), Wl2, bl2.reshape(1, 2))


# speedup vs baseline: 1.6383x; 1.6383x over previous
import functools
import math

import jax
import jax.numpy as jnp
from jax import lax
from jax.experimental import pallas as pl
from jax.experimental.pallas import tpu as pltpu
from jax.experimental.pallas import tpu_sc as plsc

N = 10000
E = 320000
G = 64
DIN = 128
H = 64
NC = 2
NS = 16
NW = NC * NS
CH = 80
NSTEP = 125
EPAD = NW * NSTEP * CH
PADROW = N + 8
ROWS_T = 632
NPAD = ROWS_T * NS
BN = 2000
NB = N // BN
K1 = 1.0 / math.sqrt(1.0 + 1e-5)

_MESH = plsc.VectorSubcoreMesh(core_axis_name="c", subcore_axis_name="s")
_SC_PARAMS = pltpu.CompilerParams(use_tc_tiling_on_sc=False)


@functools.partial(
    pl.kernel,
    out_type=jax.ShapeDtypeStruct((NC, NPAD, 16), jnp.float32),
    mesh=_MESH,
    scratch_types=[
        pltpu.VMEM((NSTEP, CH), jnp.int32),
        pltpu.VMEM((CH, 16), jnp.float32),
        pltpu.VMEM((ROWS_T, 16), jnp.float32),
        pltpu.VMEM_SHARED((NPAD, 16), jnp.float32),
    ],
    compiler_params=_SC_PARAMS,
)
def _sc_degree(dst_hbm, out_hbm, dst_v, ones_v, zero_v, acc):
    c = lax.axis_index("c")
    s = lax.axis_index("s")
    wid = c * NS + s

    def _ones(r, carry):
        ones_v[r, pl.ds(0, 16)] = jnp.ones((16,), jnp.float32)
        return carry

    def _zero(r, carry):
        zero_v[r, pl.ds(0, 16)] = jnp.zeros((16,), jnp.float32)
        return carry

    lax.fori_loop(0, CH, _ones, 0)
    lax.fori_loop(0, ROWS_T, _zero, 0)
    pltpu.sync_copy(zero_v, acc.at[pl.ds(s * ROWS_T, ROWS_T)])
    pltpu.sync_copy(dst_hbm.at[wid], dst_v)
    plsc.subcore_barrier()

    def _step(i, carry):
        pltpu.sync_copy(ones_v, acc.at[dst_v.at[i]], add=True)
        return carry

    lax.fori_loop(0, NSTEP, _step, 0)
    plsc.subcore_barrier()
    pltpu.sync_copy(acc.at[pl.ds(s * ROWS_T, ROWS_T)],
                    out_hbm.at[c, pl.ds(s * ROWS_T, ROWS_T)])


@functools.partial(
    pl.kernel,
    out_type=jax.ShapeDtypeStruct((NC, NPAD, H), jnp.float32),
    mesh=_MESH,
    scratch_types=[
        pltpu.VMEM((NSTEP, CH), jnp.int32),
        pltpu.VMEM((NSTEP, CH), jnp.int32),
        pltpu.VMEM((CH, H), jnp.float32),
        pltpu.VMEM((ROWS_T, H), jnp.float32),
        pltpu.VMEM_SHARED((NPAD, H), jnp.float32),
        pltpu.SemaphoreType.DMA,
    ],
    compiler_params=_SC_PARAMS,
)
def _sc_gather_scatter(y_hbm, src_hbm, dst_hbm, out_hbm,
                       src_v, dst_v, rows, zero_v, acc, sem):
    c = lax.axis_index("c")
    s = lax.axis_index("s")
    wid = c * NS + s

    def _zero(r, carry):
        for k in range(H // 16):
            zero_v[r, pl.ds(16 * k, 16)] = jnp.zeros((16,), jnp.float32)
        return carry

    lax.fori_loop(0, ROWS_T, _zero, 0)
    pltpu.sync_copy(zero_v, acc.at[pl.ds(s * ROWS_T, ROWS_T)])
    pltpu.sync_copy(src_hbm.at[wid], src_v)
    pltpu.sync_copy(dst_hbm.at[wid], dst_v)
    plsc.subcore_barrier()

    def _step(i, carry):
        pltpu.async_copy(y_hbm.at[src_v.at[i]], rows, sem).wait()
        pltpu.sync_copy(rows, acc.at[dst_v.at[i]], add=True)
        return carry

    lax.fori_loop(0, NSTEP, _step, 0)
    plsc.subcore_barrier()
    pltpu.sync_copy(acc.at[pl.ds(s * ROWS_T, ROWS_T)],
                    out_hbm.at[c, pl.ds(s * ROWS_T, ROWS_T)])


def _tc_pre_body(deg_ref, x_ref, w_ref, y_ref, dinv_ref):
    deg = deg_ref[0] + deg_ref[1] + 1.0
    dinv = lax.rsqrt(deg[:, :1])
    xw = jnp.dot(x_ref[...], w_ref[...], preferred_element_type=jnp.float32)
    y_ref[...] = xw * dinv
    dinv_ref[...] = jnp.broadcast_to(dinv, (BN, H))


def _tc_pre(deg, x, w1):
    return pl.pallas_call(
        _tc_pre_body,
        grid=(NB,),
        in_specs=[
            pl.BlockSpec((NC, BN, 16), lambda i: (0, i, 0)),
            pl.BlockSpec((BN, DIN), lambda i: (i, 0)),
            pl.BlockSpec((DIN, H), lambda i: (0, 0)),
        ],
        out_specs=[
            pl.BlockSpec((BN, H), lambda i: (i, 0)),
            pl.BlockSpec((BN, H), lambda i: (i, 0)),
        ],
        out_shape=[
            jax.ShapeDtypeStruct((N, H), jnp.float32),
            jax.ShapeDtypeStruct((N, H), jnp.float32),
        ],
    )(deg, x, w1)


def _make_tc_layer(has_res):
    def body(parts_ref, y_ref, dinv_ref, *rest):
        if has_res:
            res_ref, b_ref, g_ref, be_ref, w_ref, x_out, y_out = rest
        else:
            b_ref, g_ref, be_ref, w_ref, x_out, y_out = rest
        dinv = dinv_ref[...]
        agg = (parts_ref[0] + parts_ref[1] + y_ref[...]) * dinv
        h = (agg + b_ref[...]) * K1 * g_ref[...] + be_ref[...]
        xl = jnp.maximum(h, 0.0)
        if has_res:
            xl = xl + res_ref[...]
        x_out[...] = xl
        y_out[...] = jnp.dot(xl, w_ref[...],
                             preferred_element_type=jnp.float32) * dinv

    node = pl.BlockSpec((BN, H), lambda i: (i, 0))
    row = pl.BlockSpec((1, H), lambda i: (0, 0))
    in_specs = [pl.BlockSpec((NC, BN, H), lambda i: (0, i, 0)), node, node]
    if has_res:
        in_specs.append(node)
    in_specs += [row, row, row, pl.BlockSpec((H, H), lambda i: (0, 0))]

    def run(*args):
        return pl.pallas_call(
            body,
            grid=(NB,),
            in_specs=in_specs,
            out_specs=[node, node],
            out_shape=[
                jax.ShapeDtypeStruct((N, H), jnp.float32),
                jax.ShapeDtypeStruct((N, H), jnp.float32),
            ],
        )(*args)

    return run


_tc_layer_nores = _make_tc_layer(False)
_tc_layer_res = _make_tc_layer(True)


def _tc_final_body(parts_ref, y_ref, dinv_ref, b_ref, batch_ref,
                   wl1_ref, bl1_ref, wl2_ref, bl2_ref, out_ref, acc):
    i = pl.program_id(0)
    x4 = (parts_ref[0] + parts_ref[1] + y_ref[...]) * dinv_ref[...] + b_ref[...]
    bvec = batch_ref[0, 0, :]
    oht = (lax.broadcasted_iota(jnp.int32, (G, BN), 0)
           == bvec[None, :]).astype(jnp.float32)
    aug = jnp.concatenate([x4, jnp.ones_like(x4)], axis=1)
    part = jnp.dot(oht, aug, preferred_element_type=jnp.float32)
    acc[...] = jnp.where(i == 0, part, acc[...] + part)

    @pl.when(i == NB - 1)
    def _():
        sums = acc[:, :H]
        cnt = acc[:, H:H + 1]
        pooled = sums / jnp.maximum(cnt, 1.0)
        hmid = jnp.maximum(
            jnp.dot(pooled, wl1_ref[...], preferred_element_type=jnp.float32)
            + bl1_ref[...], 0.0)
        logits = jnp.dot(hmid, wl2_ref[...],
                         preferred_element_type=jnp.float32) + bl2_ref[...]
        m = jnp.max(logits, axis=1, keepdims=True)
        lse = m + jnp.log(jnp.sum(jnp.exp(logits - m), axis=1, keepdims=True))
        out_ref[...] = logits - lse


def _tc_final(parts, y4, dinv, b4, batch3, wl1, bl1, wl2, bl2):
    node = pl.BlockSpec((BN, H), lambda i: (i, 0))
    return pl.pallas_call(
        _tc_final_body,
        grid=(NB,),
        in_specs=[
            pl.BlockSpec((NC, BN, H), lambda i: (0, i, 0)),
            node, node,
            pl.BlockSpec((1, H), lambda i: (0, 0)),
            pl.BlockSpec((1, 1, BN), lambda i: (i, 0, 0)),
            pl.BlockSpec((H, H // 2), lambda i: (0, 0)),
            pl.BlockSpec((1, H // 2), lambda i: (0, 0)),
            pl.BlockSpec((H // 2, 2), lambda i: (0, 0)),
            pl.BlockSpec((1, 2), lambda i: (0, 0)),
        ],
        out_specs=pl.BlockSpec((G, 2), lambda i: (0, 0)),
        out_shape=jax.ShapeDtypeStruct((G, 2), jnp.float32),
        scratch_shapes=[pltpu.VMEM((G, 2 * H), jnp.float32)],
    )(parts, y4, dinv, b4, batch3, wl1, bl1, wl2, bl2)


def kernel(x, edge_index, batch, W1, b1, W2, b2, W3, b3, W4, b4,
           g1, be1, g2, be2, g3, be3, Wl1, bl1, Wl2, bl2):
    src = edge_index[0].astype(jnp.int32).reshape(NW, NSTEP, CH)
    dst = edge_index[1].astype(jnp.int32).reshape(NW, NSTEP, CH)
    batch3 = batch.astype(jnp.int32).reshape(NB, 1, BN)
    b1r, g1r, be1r = b1.reshape(1, H), g1.reshape(1, H), be1.reshape(1, H)
    b2r, g2r, be2r = b2.reshape(1, H), g2.reshape(1, H), be2.reshape(1, H)
    b3r, g3r, be3r = b3.reshape(1, H), g3.reshape(1, H), be3.reshape(1, H)
    b4r = b4.reshape(1, H)

    deg = _sc_degree(dst)
    y1, dinv = _tc_pre(deg, x, W1)
    p1 = _sc_gather_scatter(y1, src, dst)
    x1, y2 = _tc_layer_nores(p1, y1, dinv, b1r, g1r, be1r, W2)
    p2 = _sc_gather_scatter(y2, src, dst)
    x2, y3 = _tc_layer_res(p2, y2, dinv, x1, b2r, g2r, be2r, W3)
    p3 = _sc_gather_scatter(y3, src, dst)
    x3, y4 = _tc_layer_res(p3, y3, dinv, x2, b3r, g3r, be3r, W4)
    p4 = _sc_gather_scatter(y4, src, dst)
    return _tc_final(p4, y4, dinv, b4r, batch3,
                     Wl1, bl1.reshape(1, H // 2), Wl2, bl2.reshape(1, 2))


# speedup vs baseline: 1.6403x; 1.0012x over previous
import functools
import math

import jax
import jax.numpy as jnp
from jax import lax
from jax.experimental import pallas as pl
from jax.experimental.pallas import tpu as pltpu
from jax.experimental.pallas import tpu_sc as plsc

N = 10000
E = 320000
G = 64
DIN = 128
H = 64
NC = 2
NS = 16
NW = NC * NS
CH = 80
NSTEP = 125
EPAD = NW * NSTEP * CH
PADROW = N + 8
ROWS_T = 632
NPAD = ROWS_T * NS
BN = 2000
NB = N // BN
K1 = 1.0 / math.sqrt(1.0 + 1e-5)

_MESH = plsc.VectorSubcoreMesh(core_axis_name="c", subcore_axis_name="s")
_SC_PARAMS = pltpu.CompilerParams(use_tc_tiling_on_sc=False)


@functools.partial(
    pl.kernel,
    out_type=jax.ShapeDtypeStruct((NC, NPAD, 16), jnp.float32),
    mesh=_MESH,
    scratch_types=[
        pltpu.VMEM((NSTEP, CH), jnp.int32),
        pltpu.VMEM((CH, 16), jnp.float32),
        pltpu.VMEM((ROWS_T, 16), jnp.float32),
        pltpu.VMEM_SHARED((NPAD, 16), jnp.float32),
    ],
    compiler_params=_SC_PARAMS,
)
def _sc_degree(dst_hbm, out_hbm, dst_v, ones_v, zero_v, acc):
    c = lax.axis_index("c")
    s = lax.axis_index("s")
    wid = c * NS + s

    def _ones(r, carry):
        ones_v[r, pl.ds(0, 16)] = jnp.ones((16,), jnp.float32)
        return carry

    def _zero(r, carry):
        zero_v[r, pl.ds(0, 16)] = jnp.zeros((16,), jnp.float32)
        return carry

    lax.fori_loop(0, CH, _ones, 0)
    lax.fori_loop(0, ROWS_T, _zero, 0)
    pltpu.sync_copy(zero_v, acc.at[pl.ds(s * ROWS_T, ROWS_T)])
    pltpu.sync_copy(dst_hbm.at[wid], dst_v)
    plsc.subcore_barrier()

    def _step(i, carry):
        pltpu.sync_copy(ones_v, acc.at[dst_v.at[i]], add=True)
        return carry

    lax.fori_loop(0, NSTEP, _step, 0)
    plsc.subcore_barrier()
    pltpu.sync_copy(acc.at[pl.ds(s * ROWS_T, ROWS_T)],
                    out_hbm.at[c, pl.ds(s * ROWS_T, ROWS_T)])


@functools.partial(
    pl.kernel,
    out_type=jax.ShapeDtypeStruct((NC, NPAD, H), jnp.float32),
    mesh=_MESH,
    scratch_types=[
        pltpu.VMEM((NSTEP, CH), jnp.int32),
        pltpu.VMEM((NSTEP, CH), jnp.int32),
        pltpu.VMEM((CH, H), jnp.float32),
        pltpu.VMEM((ROWS_T, H), jnp.float32),
        pltpu.VMEM_SHARED((NPAD, H), jnp.float32),
        pltpu.SemaphoreType.DMA,
    ],
    compiler_params=_SC_PARAMS,
)
def _sc_gather_scatter(y_hbm, src_hbm, dst_hbm, out_hbm,
                       src_v, dst_v, rows, zero_v, acc, sem):
    c = lax.axis_index("c")
    s = lax.axis_index("s")
    wid = c * NS + s

    def _zero(r, carry):
        for k in range(H // 16):
            zero_v[r, pl.ds(16 * k, 16)] = jnp.zeros((16,), jnp.float32)
        return carry

    lax.fori_loop(0, ROWS_T, _zero, 0)
    pltpu.sync_copy(zero_v, acc.at[pl.ds(s * ROWS_T, ROWS_T)])
    pltpu.sync_copy(src_hbm.at[wid], src_v)
    pltpu.sync_copy(dst_hbm.at[wid], dst_v)
    plsc.subcore_barrier()

    @plsc.parallel_loop(0, NSTEP, step=1, unroll=2)
    def _step(i):
        pltpu.async_copy(y_hbm.at[src_v.at[i]], rows, sem).wait()
        pltpu.sync_copy(rows, acc.at[dst_v.at[i]], add=True)
    plsc.subcore_barrier()
    pltpu.sync_copy(acc.at[pl.ds(s * ROWS_T, ROWS_T)],
                    out_hbm.at[c, pl.ds(s * ROWS_T, ROWS_T)])


def _tc_pre_body(deg_ref, x_ref, w_ref, y_ref, dinv_ref):
    deg = deg_ref[0] + deg_ref[1] + 1.0
    dinv = lax.rsqrt(deg[:, :1])
    xw = jnp.dot(x_ref[...], w_ref[...], preferred_element_type=jnp.float32)
    y_ref[...] = xw * dinv
    dinv_ref[...] = jnp.broadcast_to(dinv, (BN, H))


def _tc_pre(deg, x, w1):
    return pl.pallas_call(
        _tc_pre_body,
        grid=(NB,),
        in_specs=[
            pl.BlockSpec((NC, BN, 16), lambda i: (0, i, 0)),
            pl.BlockSpec((BN, DIN), lambda i: (i, 0)),
            pl.BlockSpec((DIN, H), lambda i: (0, 0)),
        ],
        out_specs=[
            pl.BlockSpec((BN, H), lambda i: (i, 0)),
            pl.BlockSpec((BN, H), lambda i: (i, 0)),
        ],
        out_shape=[
            jax.ShapeDtypeStruct((N, H), jnp.float32),
            jax.ShapeDtypeStruct((N, H), jnp.float32),
        ],
    )(deg, x, w1)


def _make_tc_layer(has_res):
    def body(parts_ref, y_ref, dinv_ref, *rest):
        if has_res:
            res_ref, b_ref, g_ref, be_ref, w_ref, x_out, y_out = rest
        else:
            b_ref, g_ref, be_ref, w_ref, x_out, y_out = rest
        dinv = dinv_ref[...]
        agg = (parts_ref[0] + parts_ref[1] + y_ref[...]) * dinv
        h = (agg + b_ref[...]) * K1 * g_ref[...] + be_ref[...]
        xl = jnp.maximum(h, 0.0)
        if has_res:
            xl = xl + res_ref[...]
        x_out[...] = xl
        y_out[...] = jnp.dot(xl, w_ref[...],
                             preferred_element_type=jnp.float32) * dinv

    node = pl.BlockSpec((BN, H), lambda i: (i, 0))
    row = pl.BlockSpec((1, H), lambda i: (0, 0))
    in_specs = [pl.BlockSpec((NC, BN, H), lambda i: (0, i, 0)), node, node]
    if has_res:
        in_specs.append(node)
    in_specs += [row, row, row, pl.BlockSpec((H, H), lambda i: (0, 0))]

    def run(*args):
        return pl.pallas_call(
            body,
            grid=(NB,),
            in_specs=in_specs,
            out_specs=[node, node],
            out_shape=[
                jax.ShapeDtypeStruct((N, H), jnp.float32),
                jax.ShapeDtypeStruct((N, H), jnp.float32),
            ],
        )(*args)

    return run


_tc_layer_nores = _make_tc_layer(False)
_tc_layer_res = _make_tc_layer(True)


def _tc_final_body(parts_ref, y_ref, dinv_ref, b_ref, batch_ref,
                   wl1_ref, bl1_ref, wl2_ref, bl2_ref, out_ref, acc):
    i = pl.program_id(0)
    x4 = (parts_ref[0] + parts_ref[1] + y_ref[...]) * dinv_ref[...] + b_ref[...]
    bvec = batch_ref[0, 0, :]
    oht = (lax.broadcasted_iota(jnp.int32, (G, BN), 0)
           == bvec[None, :]).astype(jnp.float32)
    aug = jnp.concatenate([x4, jnp.ones_like(x4)], axis=1)
    part = jnp.dot(oht, aug, preferred_element_type=jnp.float32)
    acc[...] = jnp.where(i == 0, part, acc[...] + part)

    @pl.when(i == NB - 1)
    def _():
        sums = acc[:, :H]
        cnt = acc[:, H:H + 1]
        pooled = sums / jnp.maximum(cnt, 1.0)
        hmid = jnp.maximum(
            jnp.dot(pooled, wl1_ref[...], preferred_element_type=jnp.float32)
            + bl1_ref[...], 0.0)
        logits = jnp.dot(hmid, wl2_ref[...],
                         preferred_element_type=jnp.float32) + bl2_ref[...]
        m = jnp.max(logits, axis=1, keepdims=True)
        lse = m + jnp.log(jnp.sum(jnp.exp(logits - m), axis=1, keepdims=True))
        out_ref[...] = logits - lse


def _tc_final(parts, y4, dinv, b4, batch3, wl1, bl1, wl2, bl2):
    node = pl.BlockSpec((BN, H), lambda i: (i, 0))
    return pl.pallas_call(
        _tc_final_body,
        grid=(NB,),
        in_specs=[
            pl.BlockSpec((NC, BN, H), lambda i: (0, i, 0)),
            node, node,
            pl.BlockSpec((1, H), lambda i: (0, 0)),
            pl.BlockSpec((1, 1, BN), lambda i: (i, 0, 0)),
            pl.BlockSpec((H, H // 2), lambda i: (0, 0)),
            pl.BlockSpec((1, H // 2), lambda i: (0, 0)),
            pl.BlockSpec((H // 2, 2), lambda i: (0, 0)),
            pl.BlockSpec((1, 2), lambda i: (0, 0)),
        ],
        out_specs=pl.BlockSpec((G, 2), lambda i: (0, 0)),
        out_shape=jax.ShapeDtypeStruct((G, 2), jnp.float32),
        scratch_shapes=[pltpu.VMEM((G, 2 * H), jnp.float32)],
    )(parts, y4, dinv, b4, batch3, wl1, bl1, wl2, bl2)


def kernel(x, edge_index, batch, W1, b1, W2, b2, W3, b3, W4, b4,
           g1, be1, g2, be2, g3, be3, Wl1, bl1, Wl2, bl2):
    src = edge_index[0].astype(jnp.int32).reshape(NW, NSTEP, CH)
    dst = edge_index[1].astype(jnp.int32).reshape(NW, NSTEP, CH)
    batch3 = batch.astype(jnp.int32).reshape(NB, 1, BN)
    b1r, g1r, be1r = b1.reshape(1, H), g1.reshape(1, H), be1.reshape(1, H)
    b2r, g2r, be2r = b2.reshape(1, H), g2.reshape(1, H), be2.reshape(1, H)
    b3r, g3r, be3r = b3.reshape(1, H), g3.reshape(1, H), be3.reshape(1, H)
    b4r = b4.reshape(1, H)

    deg = _sc_degree(dst)
    y1, dinv = _tc_pre(deg, x, W1)
    p1 = _sc_gather_scatter(y1, src, dst)
    x1, y2 = _tc_layer_nores(p1, y1, dinv, b1r, g1r, be1r, W2)
    p2 = _sc_gather_scatter(y2, src, dst)
    x2, y3 = _tc_layer_res(p2, y2, dinv, x1, b2r, g2r, be2r, W3)
    p3 = _sc_gather_scatter(y3, src, dst)
    x3, y4 = _tc_layer_res(p3, y3, dinv, x2, b3r, g3r, be3r, W4)
    p4 = _sc_gather_scatter(y4, src, dst)
    return _tc_final(p4, y4, dinv, b4r, batch3,
                     Wl1, bl1.reshape(1, H // 2), Wl2, bl2.reshape(1, 2))
